# Initial kernel scaffold; baseline (speedup 1.0000x reference)
import jax
import jax.numpy as jnp
from jax.experimental import pallas as pl


def kernel(yi, yi_t):
    raise NotImplementedError("write your pallas kernel here")


# speedup vs baseline: 7.2683x; 7.2683x over previous
import functools

import jax
import jax.numpy as jnp
from jax.experimental import pallas as pl
from jax.experimental.pallas import tpu as pltpu

_T = 0.0025
_M = 0.6
_K = 16
_N = 256


def _tc_body(yi_ref, yit_ref, out_ref):
    yi = yi_ref[...]
    yit = yit_ref[...]
    yin = yi * jax.lax.rsqrt(jnp.sum(yi * yi, axis=1, keepdims=True) + 1e-12)
    yitn = yit * jax.lax.rsqrt(jnp.sum(yit * yit, axis=1, keepdims=True) + 1e-12)
    g = jax.lax.dot_general(yin, yin, (((1,), (1,)), ((), ())),
                            preferred_element_type=jnp.float32)
    c = jax.lax.dot_general(yitn, yin, (((1,), (1,)), ((), ())),
                            preferred_element_type=jnp.float32)
    dt = 0.5 * jnp.sqrt(jnp.maximum(2.0 - 2.0 * c, 0.0) + 1e-12)
    cols = jax.lax.broadcasted_iota(jnp.int32, (_N, _N), 1)
    rows = jax.lax.broadcasted_iota(jnp.int32, (_N, _N), 0)
    s = g
    acc = jnp.zeros((_N, 1), jnp.float32)
    d1 = jnp.zeros((_N, 1), jnp.float32)
    for t in range(_K + 1):
        m = jnp.max(s, axis=1, keepdims=True)
        hit = s == m
        amax = jnp.min(jnp.where(hit, cols, _N), axis=1, keepdims=True)
        onehot = cols == amax
        if t >= 1:
            d_t = 0.5 * jnp.sqrt(jnp.maximum(2.0 - 2.0 * m, 0.0) + 1e-12)
            dtv = jnp.sum(jnp.where(onehot, dt, 0.0), axis=1, keepdims=True)
            acc = acc + (d_t - dtv) ** 2 - _T
            if t == 1:
                d1 = d_t
        s = jnp.where(onehot, -3.0, s)
    dtt = jnp.sum(jnp.where(cols == rows, dt, 0.0), axis=1, keepdims=True)
    e2v = jnp.maximum(dtt + _M - d1, 0.0)
    total = jnp.sum(acc) + jnp.sum(e2v)
    out_ref[...] = jnp.broadcast_to(total, (1, 1))


@jax.jit
def kernel(yi, yi_t):
    out = pl.pallas_call(
        _tc_body,
        out_shape=jax.ShapeDtypeStruct((1, 1), jnp.float32),
    )(yi, yi_t)
    return out[0, 0]
